# baseline (speedup 1.0000x reference)
import functools

import jax
import jax.numpy as jnp
from jax import lax
from jax.experimental import pallas as pl
from jax.experimental.pallas import tpu as pltpu
from jax.experimental.pallas import tpu_sc as plsc

N = 10000
E = 160000
F = 256
HEADS = 8
DA = 24
DV = 32
NEG_SLOPE = 0.1

NC = 2
NS = 16
L = 16

NR = 96
RS = 112
NP = NR * RS
W = 32
EB = 2048

SROW = 640
QROW = 192
PROW = 352


def _tca_body(x_ref, w_ref, s_ref, q_ref):
    y = jnp.dot(x_ref[...], w_ref[...], preferred_element_type=jnp.float32)
    s_ref[...] = y[:, 0:640]
    q_ref[...] = y[:, 640:832]


def _tc_a(xp, wbig):
    blk = 1344
    grid = (NP // blk,)
    return pl.pallas_call(
        _tca_body,
        grid=grid,
        in_specs=[
            pl.BlockSpec((blk, F), lambda i: (i, 0)),
            pl.BlockSpec((F, 832), lambda i: (0, 0)),
        ],
        out_specs=[
            pl.BlockSpec((blk, SROW), lambda i: (i, 0)),
            pl.BlockSpec((blk, QROW), lambda i: (i, 0)),
        ],
        out_shape=[
            jax.ShapeDtypeStruct((NP, SROW), jnp.float32),
            jax.ShapeDtypeStruct((NP, QROW), jnp.float32),
        ],
    )(xp, wbig)


def _sc_body(dst_ref, src_ref, bnd_ref, s_ref, q_ref, p_ref,
             bnd_v, idxd_v, idxs_v, srow0_v, srow1_v, q_v, stage_v, sem0, sem1):
    c = lax.axis_index("c")
    s = lax.axis_index("s")
    wid = c * NS + s
    pltpu.sync_copy(bnd_ref, bnd_v)
    lane = lax.broadcasted_iota(jnp.int32, (L,), 0)
    zeros = jnp.zeros((L,), jnp.float32)
    ones = jnp.ones((L,), jnp.float32)
    neginf = jnp.full((L,), -3e38, jnp.float32)
    srow = (srow0_v, srow1_v)
    sem = (sem0, sem1)

    def range_body(p, _):
        r = wid * 3 + p
        lo = r * RS
        brow = bnd_v[pl.ds(r * L, L)]
        e_lo = brow[0]
        e_hi = brow[1]

        def zr(i, _):
            for t in range(PROW // L):
                init = neginf if t in (17, 18, 19, 20) else zeros
                stage_v[pl.ds(i * PROW + t * L, L)] = init
            return 0
        lax.fori_loop(0, RS, zr, 0)

        pltpu.sync_copy(q_ref.at[pl.ds(lo, RS)], q_v)

        a_lo = (e_lo // W) * W
        nchunks = (e_hi - a_lo + EB - 1) // EB

        def chunk_body(ci, carry):
            cstart0 = a_lo + ci * EB
            cstart = jnp.minimum(cstart0, E - EB)
            cl = jnp.maximum(e_lo, cstart0)
            cu = jnp.minimum(e_hi, cstart0 + EB)
            pltpu.sync_copy(dst_ref.at[pl.ds(cstart, EB)], idxd_v)
            pltpu.sync_copy(src_ref.at[pl.ds(cstart, EB)], idxs_v)
            wlo = (cl - cstart) // W
            k0 = (cu - cstart + W - 1) // W - wlo
            kpad = k0 + (k0 & 1)
            wcap = EB // W - 1

            def wloc_of(k):
                return jnp.minimum(wlo + k, wcap) * W

            def issue(k, b):
                wl = wloc_of(k)
                pltpu.async_copy(s_ref.at[idxs_v.at[pl.ds(wl, W)]],
                                 srow[b], sem[b])

            def window(k, b, carry):
                wl = wloc_of(k)
                wstart = cstart + wl
                sv = srow[b]

                def group_body(g, carry):
                    dvec = idxd_v[pl.ds(wl + g * L, L)]
                    for jj in range(L):
                        carry = edge_step(g * L + jj, dvec[jj], carry)
                    return carry

                def edge_step(j, d, carry):
                    e = wstart + j
                    own = (e >= cl) & (e < cu) & (k < k0)
                    dloc = jnp.clip(d - lo, 0, RS - 1)
                    own_v = lane < jnp.where(own, L, 0)
                    ownf = jnp.where(own_v, ones, zeros)
                    row = dloc * PROW

                    ss = zeros
                    for t in range(12):
                        ss = ss + q_v[dloc, pl.ds(t * L, L)] * sv[j, pl.ds(t * L, L)]
                    b_ = []
                    for h in range(HEADS):
                        lh = ss[h] + ss[h + 8]
                        b_.append(jnp.exp(jnp.full((L,), lh)) * ownf)

                    for t in range(16):
                        acc = stage_v[pl.ds(row + t * L, L)]
                        stage_v[pl.ds(row + t * L, L)] = (
                            acc + b_[t // 2] * sv[j, pl.ds(192 + t * L, L)])

                    numv = zeros
                    for h in range(HEADS):
                        numv = numv + jnp.where(lane == h, b_[h], zeros)
                    uv = sv[j, pl.ds(504, L)]
                    misc_c = jnp.where(lane < 8, numv, uv * ownf)
                    stage_v[pl.ds(row + 256, L)] = (
                        stage_v[pl.ds(row + 256, L)] + misc_c)

                    for t in range(4):
                        mv = jnp.where(own_v, sv[j, pl.ds(448 + t * L, L)], neginf)
                        zacc = stage_v[pl.ds(row + 272 + t * L, L)]
                        stage_v[pl.ds(row + 272 + t * L, L)] = jnp.maximum(zacc, mv)

                    degc = jnp.where(lane == 0, ownf, zeros)
                    stage_v[pl.ds(row + 336, L)] = (
                        stage_v[pl.ds(row + 336, L)] + degc)
                    return carry

                return lax.fori_loop(0, W // L, group_body, carry)

            issue(0, 0)

            def pair_body(kp, carry):
                for bb in range(2):
                    k = kp * 2 + bb
                    issue(jnp.minimum(k + 1, kpad - 1), (bb + 1) % 2)
                    pltpu.make_async_copy(s_ref.at[pl.ds(0, W)], srow[bb],
                                          sem[bb]).wait()
                    carry = window(k, bb, carry)
                return carry

            carry = lax.fori_loop(0, kpad // 2, pair_body, carry)
            pltpu.make_async_copy(s_ref.at[pl.ds(0, W)], srow[0], sem[0]).wait()
            return carry

        lax.fori_loop(0, nchunks, chunk_body, 0)

        pltpu.sync_copy(stage_v, p_ref.at[pl.ds(lo * PROW, RS * PROW)])
        return 0

    lax.fori_loop(0, 3, range_body, 0)


def _sc_call(dst_s, src_s, bounds, s_tab, q_tab):
    mesh = plsc.VectorSubcoreMesh(core_axis_name="c", subcore_axis_name="s",
                                  num_cores=NC, num_subcores=NS)
    f = pl.kernel(
        _sc_body,
        out_type=jax.ShapeDtypeStruct((NP * PROW,), jnp.float32),
        mesh=mesh,
        scratch_types=[
            pltpu.VMEM((NR * L,), jnp.int32),
            pltpu.VMEM((EB,), jnp.int32),
            pltpu.VMEM((EB,), jnp.int32),
            pltpu.VMEM((W, SROW), jnp.float32),
            pltpu.VMEM((W, SROW), jnp.float32),
            pltpu.VMEM((RS, QROW), jnp.float32),
            pltpu.VMEM((RS * PROW,), jnp.float32),
            pltpu.SemaphoreType.DMA,
            pltpu.SemaphoreType.DMA,
        ],
    )
    return f(dst_s, src_s, bounds, s_tab, q_tab).reshape(NP, PROW)


def _tcb_body(x_ref, p_ref, wg_ref, bg_ref, wo_ref, bo_ref, o_ref, *, mode):
    x = x_ref[...]
    p = p_ref[...]
    aggu = p[:, 0:256]
    den = p[:, 256:264]
    usum = p[:, 264:272]
    zmax = p[:, 272:336]
    deg = p[:, 336:337]
    zmax = jnp.where(deg > 0, zmax, 0.0)
    wg = wg_ref[...]
    glin = (jnp.dot(x, wg[0:256], preferred_element_type=jnp.float32)
            + jnp.dot(zmax, wg[256:320], preferred_element_type=jnp.float32)
            + usum / jnp.maximum(deg, 1.0)
            + bg_ref[...][None, :])
    gate = jax.nn.sigmoid(glin)
    coef = gate / (den + 1e-30)
    rows = lax.broadcasted_iota(jnp.int32, (8, 256), 0)
    colh = lax.broadcasted_iota(jnp.int32, (8, 256), 1) // 32
    expander = (rows == colh).astype(jnp.float32)
    gated = aggu * jnp.dot(coef, expander, preferred_element_type=jnp.float32)
    wo = wo_ref[...]
    out = (jnp.dot(x, wo[0:256], preferred_element_type=jnp.float32)
           + jnp.dot(gated, wo[256:512], preferred_element_type=jnp.float32)
           + bo_ref[...][None, :])
    if mode == "leaky":
        out = jnp.where(out >= 0, out, out * NEG_SLOPE)
    else:
        m = jnp.max(out, axis=-1, keepdims=True)
        zz = out - m
        out = zz - jnp.log(jnp.sum(jnp.exp(zz), axis=-1, keepdims=True))
    o_ref[...] = out


def _tc_b(xp, p, wg, bg, wo, bo, mode, out_c):
    blk = 1344
    grid = (NP // blk,)
    body = functools.partial(_tcb_body, mode=mode)
    return pl.pallas_call(
        body,
        grid=grid,
        in_specs=[
            pl.BlockSpec((blk, F), lambda i: (i, 0)),
            pl.BlockSpec((blk, PROW), lambda i: (i, 0)),
            pl.BlockSpec((576, HEADS), lambda i: (0, 0)),
            pl.BlockSpec((HEADS,), lambda i: (0,)),
            pl.BlockSpec((512, out_c), lambda i: (0, 0)),
            pl.BlockSpec((out_c,), lambda i: (0,)),
        ],
        out_specs=pl.BlockSpec((blk, out_c), lambda i: (i, 0)),
        out_shape=jax.ShapeDtypeStruct((NP, out_c), jnp.float32),
    )(xp, p, wg, bg, wo, bo)


def _wbig(Wq, Wk, Wv, Wm, Wg):
    k_r = Wk.reshape(F, HEADS, DA).transpose(0, 2, 1).reshape(F, 192)
    q_r = Wq.reshape(F, HEADS, DA).transpose(0, 2, 1).reshape(F, 192)
    wg3 = Wg[320:576]
    pad = jnp.zeros((F, 120), jnp.float32)
    return jnp.concatenate([k_r, Wv, Wm, wg3, pad, q_r], axis=1)


def _layer(xp, dst_s, src_s, bounds, Wq, Wk, Wv, Wm, Wg, bg, Wo, bo, mode, out_c):
    wbig = _wbig(Wq, Wk, Wv, Wm, Wg)
    s_tab, q_tab = _tc_a(xp, wbig)
    p = _sc_call(dst_s, src_s, bounds, s_tab, q_tab)
    return _tc_b(xp, p, Wg, bg, Wo, bo, mode, out_c)


def kernel(x, adjs, l0_Wq, l0_Wk, l0_Wv, l0_Wm, l0_Wg, l0_bg, l0_Wo, l0_bo,
           l1_Wq, l1_Wk, l1_Wv, l1_Wm, l1_Wg, l1_bg, l1_Wo, l1_bo):
    src = adjs[0]
    dst = adjs[1]
    dst_s, src_s = lax.sort((dst, src), num_keys=1)
    starts = (jnp.arange(NR + 1, dtype=jnp.int32) * RS).clip(max=N)
    b = jnp.searchsorted(dst_s, starts).astype(jnp.int32)
    bounds = jnp.pad(jnp.stack([b[:NR], b[1:NR + 1]], axis=1),
                     ((0, 0), (0, L - 2))).reshape(NR * L)
    xp = jnp.pad(x, ((0, NP - N), (0, 0)))
    h = _layer(xp, dst_s, src_s, bounds, l0_Wq, l0_Wk, l0_Wv, l0_Wm, l0_Wg,
               l0_bg, l0_Wo, l0_bo, "leaky", 256)
    out = _layer(h, dst_s, src_s, bounds, l1_Wq, l1_Wk, l1_Wv, l1_Wm, l1_Wg,
                 l1_bg, l1_Wo, l1_bo, "logsoftmax", 64)
    return out[:N]


# speedup vs baseline: 2.1229x; 2.1229x over previous
import functools

import jax
import jax.numpy as jnp
from jax import lax
from jax.experimental import pallas as pl
from jax.experimental.pallas import tpu as pltpu
from jax.experimental.pallas import tpu_sc as plsc

N = 10000
E = 160000
F = 256
HEADS = 8
DA = 24
DV = 32
NEG_SLOPE = 0.1

NC = 2
NS = 16
L = 16

NR = 96
RS = 112
NP = NR * RS
W = 32
EB = 2048

SROW = 640
QROW = 192
PROW = 352


def _tca_body(x_ref, w_ref, s_ref, q_ref):
    y = jnp.dot(x_ref[...], w_ref[...], preferred_element_type=jnp.float32)
    s_ref[...] = y[:, 0:640]
    q_ref[...] = y[:, 640:832]


def _tc_a(xp, wbig):
    blk = 1344
    grid = (NP // blk,)
    return pl.pallas_call(
        _tca_body,
        grid=grid,
        in_specs=[
            pl.BlockSpec((blk, F), lambda i: (i, 0)),
            pl.BlockSpec((F, 832), lambda i: (0, 0)),
        ],
        out_specs=[
            pl.BlockSpec((blk, SROW), lambda i: (i, 0)),
            pl.BlockSpec((blk, QROW), lambda i: (i, 0)),
        ],
        out_shape=[
            jax.ShapeDtypeStruct((NP, SROW), jnp.float32),
            jax.ShapeDtypeStruct((NP, QROW), jnp.float32),
        ],
    )(xp, wbig)


def _sc_body(dst_ref, src_ref, bnd_ref, s_ref, q_ref, p_ref,
             bnd_v, idxd_v, idxs_v, srow0_v, srow1_v, q_v, stage_v, sem0, sem1):
    c = lax.axis_index("c")
    s = lax.axis_index("s")
    wid = c * NS + s
    pltpu.sync_copy(bnd_ref, bnd_v)
    lane = lax.broadcasted_iota(jnp.int32, (L,), 0)
    zeros = jnp.zeros((L,), jnp.float32)
    ones = jnp.ones((L,), jnp.float32)
    neginf = jnp.full((L,), -3e38, jnp.float32)
    srow = (srow0_v, srow1_v)
    sem = (sem0, sem1)

    def range_body(p, _):
        r = wid * 3 + p
        lo = r * RS
        brow = bnd_v[pl.ds(r * L, L)]
        e_lo = brow[0]
        e_hi = brow[1]

        def zr(i, _):
            for t in range(PROW // L):
                stage_v[pl.ds(i * PROW + t * L, L)] = zeros
            return 0
        lax.fori_loop(0, RS, zr, 0)

        pltpu.sync_copy(q_ref.at[pl.ds(lo, RS)], q_v)

        a_lo = (e_lo // W) * W
        nchunks = (e_hi - a_lo + EB - 1) // EB

        def chunk_body(ci, carry):
            cstart0 = a_lo + ci * EB
            cstart = jnp.minimum(cstart0, E - EB)
            cl = jnp.maximum(e_lo, cstart0)
            cu = jnp.minimum(e_hi, cstart0 + EB)
            pltpu.sync_copy(dst_ref.at[pl.ds(cstart, EB)], idxd_v)
            pltpu.sync_copy(src_ref.at[pl.ds(cstart, EB)], idxs_v)
            wlo = (cl - cstart) // W
            k0 = (cu - cstart + W - 1) // W - wlo
            kpad = k0 + (k0 & 1)
            wcap = EB // W - 1

            def wloc_of(k):
                return jnp.minimum(wlo + k, wcap) * W

            def issue(k, b):
                wl = wloc_of(k)
                pltpu.async_copy(s_ref.at[idxs_v.at[pl.ds(wl, W)]],
                                 srow[b], sem[b])

            def window(k, b, carry):
                wl = wloc_of(k)
                wstart = cstart + wl
                sv = srow[b]

                def group_body(g, carry):
                    dvec = idxd_v[pl.ds(wl + g * L, L)]
                    for jj in range(L):
                        carry = edge_step(g * L + jj, dvec[jj], carry)
                    return carry

                def edge_step(j, d, carry):
                    cur, aggu, misc, degv, z = carry
                    e = wstart + j
                    own = (e >= cl) & (e < cu) & (k < k0)
                    is_new = own & (d != cur)
                    dloc = jnp.clip(d - lo, 0, RS - 1)
                    own_v = lane < jnp.where(own, L, 0)
                    isn_v = lane < jnp.where(is_new, L, 0)

                    aggu = [jnp.where(isn_v, zeros, aggu[t]) for t in range(16)]
                    misc = jnp.where(isn_v, zeros, misc)
                    degv = jnp.where(isn_v, zeros, degv)
                    z = [jnp.where(isn_v, neginf, z[t]) for t in range(4)]
                    cur = jnp.where(is_new, d, cur)

                    ss = zeros
                    for t in range(12):
                        ss = ss + q_v[dloc, pl.ds(t * L, L)] * sv[j, pl.ds(t * L, L)]
                    ownf = jnp.where(own_v, ones, zeros)
                    b_ = []
                    for h in range(HEADS):
                        lh = ss[h] + ss[h + 8]
                        b_.append(jnp.exp(jnp.full((L,), lh)) * ownf)

                    for t in range(16):
                        aggu[t] = aggu[t] + b_[t // 2] * sv[j, pl.ds(192 + t * L, L)]

                    numv = zeros
                    for h in range(HEADS):
                        numv = numv + jnp.where(lane == h, b_[h], zeros)
                    uv = sv[j, pl.ds(504, L)]
                    misc = misc + jnp.where(lane < 8, numv, uv * ownf)
                    degv = degv + jnp.where(lane == 0, ownf, zeros)

                    for t in range(4):
                        mv = sv[j, pl.ds(448 + t * L, L)]
                        z[t] = jnp.where(own_v, jnp.maximum(z[t], mv), z[t])

                    row = jnp.clip(cur - lo, 0, RS - 1) * PROW
                    for t in range(16):
                        stage_v[pl.ds(row + t * L, L)] = aggu[t]
                    stage_v[pl.ds(row + 256, L)] = misc
                    for t in range(4):
                        stage_v[pl.ds(row + 272 + t * L, L)] = z[t]
                    stage_v[pl.ds(row + 336, L)] = degv
                    return (cur, aggu, misc, degv, z)

                return lax.fori_loop(0, W // L, group_body, carry)

            issue(0, 0)

            def pair_body(kp, carry):
                for bb in range(2):
                    k = kp * 2 + bb
                    issue(jnp.minimum(k + 1, kpad - 1), (bb + 1) % 2)
                    pltpu.make_async_copy(s_ref.at[pl.ds(0, W)], srow[bb],
                                          sem[bb]).wait()
                    carry = window(k, bb, carry)
                return carry

            carry = lax.fori_loop(0, kpad // 2, pair_body, carry)
            pltpu.make_async_copy(s_ref.at[pl.ds(0, W)], srow[0], sem[0]).wait()
            return carry

        init = (jnp.int32(-1), [zeros] * 16, zeros, zeros, [neginf] * 4)
        lax.fori_loop(0, nchunks, chunk_body, init)

        pltpu.sync_copy(stage_v, p_ref.at[pl.ds(lo * PROW, RS * PROW)])
        return 0

    lax.fori_loop(0, 3, range_body, 0)


def _sc_call(dst_s, src_s, bounds, s_tab, q_tab):
    mesh = plsc.VectorSubcoreMesh(core_axis_name="c", subcore_axis_name="s",
                                  num_cores=NC, num_subcores=NS)
    f = pl.kernel(
        _sc_body,
        out_type=jax.ShapeDtypeStruct((NP * PROW,), jnp.float32),
        mesh=mesh,
        scratch_types=[
            pltpu.VMEM((NR * L,), jnp.int32),
            pltpu.VMEM((EB,), jnp.int32),
            pltpu.VMEM((EB,), jnp.int32),
            pltpu.VMEM((W, SROW), jnp.float32),
            pltpu.VMEM((W, SROW), jnp.float32),
            pltpu.VMEM((RS, QROW), jnp.float32),
            pltpu.VMEM((RS * PROW,), jnp.float32),
            pltpu.SemaphoreType.DMA,
            pltpu.SemaphoreType.DMA,
        ],
    )
    return f(dst_s, src_s, bounds, s_tab, q_tab).reshape(NP, PROW)


def _tcb_body(x_ref, p_ref, wg_ref, bg_ref, wo_ref, bo_ref, o_ref, *, mode):
    x = x_ref[...]
    p = p_ref[...]
    aggu = p[:, 0:256]
    den = p[:, 256:264]
    usum = p[:, 264:272]
    zmax = p[:, 272:336]
    deg = p[:, 336:337]
    zmax = jnp.where(deg > 0, zmax, 0.0)
    wg = wg_ref[...]
    glin = (jnp.dot(x, wg[0:256], preferred_element_type=jnp.float32)
            + jnp.dot(zmax, wg[256:320], preferred_element_type=jnp.float32)
            + usum / jnp.maximum(deg, 1.0)
            + bg_ref[...][None, :])
    gate = jax.nn.sigmoid(glin)
    coef = gate / (den + 1e-30)
    rows = lax.broadcasted_iota(jnp.int32, (8, 256), 0)
    colh = lax.broadcasted_iota(jnp.int32, (8, 256), 1) // 32
    expander = (rows == colh).astype(jnp.float32)
    gated = aggu * jnp.dot(coef, expander, preferred_element_type=jnp.float32)
    wo = wo_ref[...]
    out = (jnp.dot(x, wo[0:256], preferred_element_type=jnp.float32)
           + jnp.dot(gated, wo[256:512], preferred_element_type=jnp.float32)
           + bo_ref[...][None, :])
    if mode == "leaky":
        out = jnp.where(out >= 0, out, out * NEG_SLOPE)
    else:
        m = jnp.max(out, axis=-1, keepdims=True)
        zz = out - m
        out = zz - jnp.log(jnp.sum(jnp.exp(zz), axis=-1, keepdims=True))
    o_ref[...] = out


def _tc_b(xp, p, wg, bg, wo, bo, mode, out_c):
    blk = 1344
    grid = (NP // blk,)
    body = functools.partial(_tcb_body, mode=mode)
    return pl.pallas_call(
        body,
        grid=grid,
        in_specs=[
            pl.BlockSpec((blk, F), lambda i: (i, 0)),
            pl.BlockSpec((blk, PROW), lambda i: (i, 0)),
            pl.BlockSpec((576, HEADS), lambda i: (0, 0)),
            pl.BlockSpec((HEADS,), lambda i: (0,)),
            pl.BlockSpec((512, out_c), lambda i: (0, 0)),
            pl.BlockSpec((out_c,), lambda i: (0,)),
        ],
        out_specs=pl.BlockSpec((blk, out_c), lambda i: (i, 0)),
        out_shape=jax.ShapeDtypeStruct((NP, out_c), jnp.float32),
    )(xp, p, wg, bg, wo, bo)


def _wbig(Wq, Wk, Wv, Wm, Wg):
    k_r = Wk.reshape(F, HEADS, DA).transpose(0, 2, 1).reshape(F, 192)
    q_r = Wq.reshape(F, HEADS, DA).transpose(0, 2, 1).reshape(F, 192)
    wg3 = Wg[320:576]
    pad = jnp.zeros((F, 120), jnp.float32)
    return jnp.concatenate([k_r, Wv, Wm, wg3, pad, q_r], axis=1)


def _layer(xp, dst_s, src_s, bounds, Wq, Wk, Wv, Wm, Wg, bg, Wo, bo, mode, out_c):
    wbig = _wbig(Wq, Wk, Wv, Wm, Wg)
    s_tab, q_tab = _tc_a(xp, wbig)
    p = _sc_call(dst_s, src_s, bounds, s_tab, q_tab)
    return _tc_b(xp, p, Wg, bg, Wo, bo, mode, out_c)


def kernel(x, adjs, l0_Wq, l0_Wk, l0_Wv, l0_Wm, l0_Wg, l0_bg, l0_Wo, l0_bo,
           l1_Wq, l1_Wk, l1_Wv, l1_Wm, l1_Wg, l1_bg, l1_Wo, l1_bo):
    src = adjs[0]
    dst = adjs[1]
    dst_s, src_s = lax.sort((dst, src), num_keys=1)
    starts = (jnp.arange(NR + 1, dtype=jnp.int32) * RS).clip(max=N)
    b = jnp.searchsorted(dst_s, starts).astype(jnp.int32)
    bounds = jnp.pad(jnp.stack([b[:NR], b[1:NR + 1]], axis=1),
                     ((0, 0), (0, L - 2))).reshape(NR * L)
    xp = jnp.pad(x, ((0, NP - N), (0, 0)))
    h = _layer(xp, dst_s, src_s, bounds, l0_Wq, l0_Wk, l0_Wv, l0_Wm, l0_Wg,
               l0_bg, l0_Wo, l0_bo, "leaky", 256)
    out = _layer(h, dst_s, src_s, bounds, l1_Wq, l1_Wk, l1_Wv, l1_Wm, l1_Wg,
                 l1_bg, l1_Wo, l1_bo, "logsoftmax", 64)
    return out[:N]
